# baseline (speedup 1.0000x reference)
import functools

import jax
import jax.numpy as jnp
from jax import lax
from jax.experimental import pallas as pl
from jax.experimental.pallas import tpu as pltpu
from jax.experimental.pallas import tpu_sc as plsc

N = 10000
E = 160000
FEAT = 256
HID = 256
OUT = 64

N_PAD = 10240
E_PAD = 163840
ROWS_PER_TILE = N // 16

_MESH = plsc.VectorSubcoreMesh(core_axis_name="c", subcore_axis_name="s")


def _deg_body(col_h, ew_h, out_h, idx_v, ew_v, zbuf, degacc):
    c = lax.axis_index("c")
    s = lax.axis_index("s")
    w = c * 16 + s
    for j in range(40):
        zbuf[pl.ds(16 * j, 16)] = jnp.zeros((16,), jnp.float32)
    pltpu.sync_copy(zbuf, degacc.at[pl.ds(640 * s, 640)])
    pltpu.sync_copy(col_h.at[w], idx_v)
    pltpu.sync_copy(ew_h.at[w], ew_v)
    plsc.subcore_barrier()

    def step(k, carry):
        pltpu.sync_copy(ew_v.at[k], degacc.at[idx_v.at[k]], add=True)
        return carry

    lax.fori_loop(0, 40, step, 0)
    plsc.subcore_barrier()
    pltpu.sync_copy(degacc.at[pl.ds(640 * s, 640)], out_h.at[c, pl.ds(640 * s, 640)])


_deg_kernel = pl.kernel(
    _deg_body,
    out_type=jax.ShapeDtypeStruct((2, N_PAD), jnp.float32),
    mesh=_MESH,
    scratch_types=[
        pltpu.VMEM((40, 128), jnp.int32),
        pltpu.VMEM((40, 128), jnp.float32),
        pltpu.VMEM((640,), jnp.float32),
        pltpu.VMEM_SHARED((N_PAD,), jnp.float32),
    ],
)


def _agg_body(D, ya, yb, row_h, col_h, ew_h, outa, outb,
              ridx, cidx, ewv, rows, rows2, acc, sem, sem2):
    c = lax.axis_index("c")
    s = lax.axis_index("s")
    base = 640 * s

    def _stripe_copy(src_at, dst_at):
        @pl.when(s < 15)
        def _():
            pltpu.sync_copy(src_at(base, 640), dst_at(base, 640))

        @pl.when(s == 15)
        def _():
            pltpu.sync_copy(src_at(9600, 400), dst_at(9600, 400))

    @pl.when(c == 0)
    def _():
        _stripe_copy(lambda b, n: ya.at[pl.ds(b, n)],
                     lambda b, n: acc.at[pl.ds(b, n)])

    @pl.when(c == 1)
    def _():
        _stripe_copy(lambda b, n: yb.at[pl.ds(b, n)],
                     lambda b, n: acc.at[pl.ds(b, n)])

    plsc.subcore_barrier()

    def _gather(k, rbuf, gsem):
        @pl.when(c == 0)
        def _():
            pltpu.async_copy(ya.at[ridx.at[k]], rbuf, gsem)

        @pl.when(c == 1)
        def _():
            pltpu.async_copy(yb.at[ridx.at[k]], rbuf, gsem)

    def _drain(rbuf, gsem):
        pltpu.make_async_copy(ya.at[pl.ds(0, 128)], rbuf, gsem).wait()

    def _scale(k, rbuf):
        def scale_group(g2, rc):
            for gg in range(2):
                g = 2 * g2 + gg
                ewvec = ewv[k, pl.ds(16 * g, 16)]
                for r in range(16):
                    wv = lax.gather(
                        ewvec, jnp.full((16, 1), r, jnp.int32),
                        lax.GatherDimensionNumbers(
                            offset_dims=(), collapsed_slice_dims=(0,),
                            start_index_map=(0,)),
                        (1,), mode=lax.GatherScatterMode.PROMISE_IN_BOUNDS)
                    row = 16 * g + r
                    for q in range(8):
                        rbuf[row, pl.ds(16 * q, 16)] = (
                            rbuf[row, pl.ds(16 * q, 16)] * wv)
            return rc

        lax.fori_loop(0, 4, scale_group, 0)

    for p in range(2):
        pltpu.sync_copy(row_h.at[s, pl.ds(40 * p, 40)], ridx)
        pltpu.sync_copy(col_h.at[s, pl.ds(40 * p, 40)], cidx)
        pltpu.sync_copy(ew_h.at[s, pl.ds(40 * p, 40)], ewv)
        _gather(0, rows, sem)
        _gather(1, rows2, sem2)

        def step(k, carry):
            for b, (rbuf, gsem) in enumerate(((rows, sem), (rows2, sem2))):
                j = 2 * k + b
                _drain(rbuf, gsem)
                _scale(j, rbuf)
                pltpu.sync_copy(rbuf, acc.at[cidx.at[j]], add=True)

                @pl.when(j + 2 < 40)
                def _():
                    _gather(j + 2, rbuf, gsem)
            return carry

        lax.fori_loop(0, 20, step, 0)
    plsc.subcore_barrier()

    @pl.when(c == 0)
    def _():
        _stripe_copy(lambda b, n: acc.at[pl.ds(b, n)],
                     lambda b, n: outa.at[pl.ds(b, n)])

    @pl.when(c == 1)
    def _():
        _stripe_copy(lambda b, n: acc.at[pl.ds(b, n)],
                     lambda b, n: outb.at[pl.ds(b, n)])


_agg128 = pl.kernel(
    functools.partial(_agg_body, 128),
    out_type=(jax.ShapeDtypeStruct((N, 128), jnp.float32),
              jax.ShapeDtypeStruct((N, 128), jnp.float32)),
    mesh=_MESH,
    scratch_types=[
        pltpu.VMEM((40, 128), jnp.int32),
        pltpu.VMEM((40, 128), jnp.int32),
        pltpu.VMEM((40, 128), jnp.float32),
        pltpu.VMEM((128, 128), jnp.float32),
        pltpu.VMEM((128, 128), jnp.float32),
        pltpu.VMEM_SHARED((N, 128), jnp.float32),
        pltpu.SemaphoreType.DMA,
        pltpu.SemaphoreType.DMA,
    ],
)


def _agg23_body(y23, row_h, col_h, ew_h, out0, out1,
                ridx, cidx, ewv, rows, rows2, acc, sem, sem2):
    c = lax.axis_index("c")
    s = lax.axis_index("s")
    w = c * 16 + s
    base = 640 * s

    def _stripe_copy(src_at, dst_at):
        @pl.when(s < 15)
        def _():
            pltpu.sync_copy(src_at(base, 640), dst_at(base, 640))

        @pl.when(s == 15)
        def _():
            pltpu.sync_copy(src_at(9600, 400), dst_at(9600, 400))

    @pl.when(c == 0)
    def _():
        _stripe_copy(lambda b, n: y23.at[pl.ds(b, n)],
                     lambda b, n: acc.at[pl.ds(b, n)])

    @pl.when(c == 1)
    def _():
        def zfill(j, carry):
            for q in range(8):
                rows[j, pl.ds(16 * q, 16)] = jnp.zeros((16,), jnp.float32)
            return carry

        lax.fori_loop(0, 128, zfill, 0)

        @pl.when(s < 15)
        def _():
            for t in range(5):
                pltpu.sync_copy(rows, acc.at[pl.ds(base + 128 * t, 128)])

        @pl.when(s == 15)
        def _():
            for t in range(3):
                pltpu.sync_copy(rows, acc.at[pl.ds(9600 + 128 * t, 128)])
            pltpu.sync_copy(rows.at[pl.ds(0, 16)], acc.at[pl.ds(9984, 16)])

    pltpu.sync_copy(row_h.at[w], ridx)
    pltpu.sync_copy(col_h.at[w], cidx)
    pltpu.sync_copy(ew_h.at[w], ewv)
    plsc.subcore_barrier()

    def _scale(k, rbuf):
        def scale_group(g2, rc):
            for gg in range(2):
                g = 2 * g2 + gg
                ewvec = ewv[k, pl.ds(16 * g, 16)]
                for r in range(16):
                    wv = lax.gather(
                        ewvec, jnp.full((16, 1), r, jnp.int32),
                        lax.GatherDimensionNumbers(
                            offset_dims=(), collapsed_slice_dims=(0,),
                            start_index_map=(0,)),
                        (1,), mode=lax.GatherScatterMode.PROMISE_IN_BOUNDS)
                    row = 16 * g + r
                    for q in range(8):
                        rbuf[row, pl.ds(16 * q, 16)] = (
                            rbuf[row, pl.ds(16 * q, 16)] * wv)
            return rc

        lax.fori_loop(0, 4, scale_group, 0)

    NB = 40
    pltpu.async_copy(y23.at[ridx.at[0]], rows, sem)
    pltpu.async_copy(y23.at[ridx.at[1]], rows2, sem2)

    def step(k, carry):
        for b, (rbuf, gsem) in enumerate(((rows, sem), (rows2, sem2))):
            j = 2 * k + b
            pltpu.make_async_copy(y23.at[pl.ds(0, 128)], rbuf, gsem).wait()
            _scale(j, rbuf)
            pltpu.sync_copy(rbuf, acc.at[cidx.at[j]], add=True)

            @pl.when(j + 2 < NB)
            def _():
                pltpu.async_copy(y23.at[ridx.at[j + 2]], rbuf, gsem)
        return carry

    lax.fori_loop(0, NB // 2, step, 0)
    plsc.subcore_barrier()

    @pl.when(c == 0)
    def _():
        _stripe_copy(lambda b, n: acc.at[pl.ds(b, n)],
                     lambda b, n: out0.at[pl.ds(b, n)])

    @pl.when(c == 1)
    def _():
        _stripe_copy(lambda b, n: acc.at[pl.ds(b, n)],
                     lambda b, n: out1.at[pl.ds(b, n)])


_agg23 = pl.kernel(
    _agg23_body,
    out_type=(jax.ShapeDtypeStruct((N, 128), jnp.float32),
              jax.ShapeDtypeStruct((N, 128), jnp.float32)),
    mesh=_MESH,
    scratch_types=[
        pltpu.VMEM((40, 128), jnp.int32),
        pltpu.VMEM((40, 128), jnp.int32),
        pltpu.VMEM((40, 128), jnp.float32),
        pltpu.VMEM((128, 128), jnp.float32),
        pltpu.VMEM((128, 128), jnp.float32),
        pltpu.VMEM_SHARED((N, 128), jnp.float32),
        pltpu.SemaphoreType.DMA,
        pltpu.SemaphoreType.DMA,
    ],
)


_PREC = lax.Precision.DEFAULT


def _dinv_body(deg_ref, out_ref):
    d = deg_ref[0] + deg_ref[1] + 1.0
    out_ref[...] = lax.rsqrt(d)


_dinv_call = pl.pallas_call(
    _dinv_body,
    out_shape=jax.ShapeDtypeStruct((N_PAD // 128, 128), jnp.float32),
)


def _mm1_body(x_ref, w_ref, dinv_ref, outa_ref, outb_ref):
    xw = lax.dot(x_ref[...], w_ref[...], precision=_PREC,
                 preferred_element_type=jnp.float32)
    y = xw * dinv_ref[...]
    outa_ref[...] = y[:, :128]
    outb_ref[...] = y[:, 128:]


_BM = 400

_mm1_call = pl.pallas_call(
    _mm1_body,
    grid=(N // _BM,),
    in_specs=[
        pl.BlockSpec((_BM, FEAT), lambda i: (i, 0)),
        pl.BlockSpec((FEAT, HID), lambda i: (0, 0)),
        pl.BlockSpec((_BM, 1), lambda i: (i, 0)),
    ],
    out_specs=[
        pl.BlockSpec((_BM, 128), lambda i: (i, 0)),
        pl.BlockSpec((_BM, 128), lambda i: (i, 0)),
    ],
    out_shape=[
        jax.ShapeDtypeStruct((N, 128), jnp.float32),
        jax.ShapeDtypeStruct((N, 128), jnp.float32),
    ],
)


def _mm2_body(acca_ref, accb_ref, dinv_ref, b1a_ref, b1b_ref,
              wa_ref, wb_ref, outa_ref):
    dinv = dinv_ref[...]
    x1a = jax.nn.sigmoid(acca_ref[...] * dinv + b1a_ref[...])
    x1b = jax.nn.sigmoid(accb_ref[...] * dinv + b1b_ref[...])
    y = lax.dot(x1a, wa_ref[...], precision=_PREC,
                preferred_element_type=jnp.float32)
    y = y + lax.dot(x1b, wb_ref[...], precision=_PREC,
                    preferred_element_type=jnp.float32)
    outa_ref[...] = y * dinv


_mm2_call = pl.pallas_call(
    _mm2_body,
    grid=(N // _BM,),
    in_specs=[
        pl.BlockSpec((_BM, 128), lambda i: (i, 0)),
        pl.BlockSpec((_BM, 128), lambda i: (i, 0)),
        pl.BlockSpec((_BM, 1), lambda i: (i, 0)),
        pl.BlockSpec((1, 128), lambda i: (0, 0)),
        pl.BlockSpec((1, 128), lambda i: (0, 0)),
        pl.BlockSpec((128, 2 * OUT), lambda i: (0, 0)),
        pl.BlockSpec((128, 2 * OUT), lambda i: (0, 0)),
    ],
    out_specs=pl.BlockSpec((_BM, 128), lambda i: (i, 0)),
    out_shape=jax.ShapeDtypeStruct((N, 128), jnp.float32),
)


def _mmout_body(p0_ref, p1_ref, dinv_ref, b2_ref, b3_ref,
                mu_ref, lv_ref):
    dinv = dinv_ref[...]
    su = p0_ref[...] + p1_ref[...]
    mu_ref[...] = jax.nn.sigmoid(su[:, :OUT] * dinv + b2_ref[...])
    lv_ref[...] = jax.nn.sigmoid(su[:, OUT:] * dinv + b3_ref[...])


_mmout_call = pl.pallas_call(
    _mmout_body,
    grid=(N // _BM,),
    in_specs=[
        pl.BlockSpec((_BM, 128), lambda i: (i, 0)),
        pl.BlockSpec((_BM, 128), lambda i: (i, 0)),
        pl.BlockSpec((_BM, 1), lambda i: (i, 0)),
        pl.BlockSpec((1, OUT), lambda i: (0, 0)),
        pl.BlockSpec((1, OUT), lambda i: (0, 0)),
    ],
    out_specs=[
        pl.BlockSpec((_BM, OUT), lambda i: (i, 0)),
        pl.BlockSpec((_BM, OUT), lambda i: (i, 0)),
    ],
    out_shape=[
        jax.ShapeDtypeStruct((N, OUT), jnp.float32),
        jax.ShapeDtypeStruct((N, OUT), jnp.float32),
    ],
)


def _adj_body(zi_ref, zj_ref, out_ref):
    p = lax.dot_general(zi_ref[...], zj_ref[...],
                        (((1,), (1,)), ((), ())),
                        precision=_PREC,
                        preferred_element_type=jnp.float32)
    out_ref[...] = jax.nn.sigmoid(p)


_BA = 512
_adj_call = pl.pallas_call(
    _adj_body,
    grid=(pl.cdiv(N, _BA), pl.cdiv(N, _BA)),
    in_specs=[
        pl.BlockSpec((_BA, OUT), lambda i, j: (i, 0)),
        pl.BlockSpec((_BA, OUT), lambda i, j: (j, 0)),
    ],
    out_specs=pl.BlockSpec((_BA, _BA), lambda i, j: (i, j)),
    out_shape=jax.ShapeDtypeStruct((N, N), jnp.float32),
)


def kernel(x, edge_index, edge_weight, W1, b1, W2, b2, W3, b3):
    row = edge_index[0].astype(jnp.int32)
    col = edge_index[1].astype(jnp.int32)
    ew = edge_weight.astype(jnp.float32)

    pad = E_PAD - E
    row_p = jnp.concatenate([row, jnp.zeros((pad,), jnp.int32)])
    col_p = jnp.concatenate([col, jnp.zeros((pad,), jnp.int32)])
    ew_p = jnp.concatenate([ew, jnp.zeros((pad,), jnp.float32)])

    row_d = row_p.reshape(32, 40, 128)
    col_d = col_p.reshape(32, 40, 128)
    ew_d = ew_p.reshape(32, 40, 128)
    row_a = row_p.reshape(16, 80, 128)
    col_a = col_p.reshape(16, 80, 128)
    ew_a = ew_p.reshape(16, 80, 128)

    deg_p = _deg_kernel(col_d, ew_d)
    dinv2d = _dinv_call(deg_p.reshape(2, N_PAD // 128, 128))
    dinv_col = dinv2d.reshape(N_PAD)[:N].reshape(N, 1)

    ya, yb = _mm1_call(x, W1, dinv_col)
    acc0, acc1 = _agg128(ya, yb, row_a, col_a, ew_a)

    W23 = jnp.concatenate([W2, W3], axis=1)
    y23 = _mm2_call(acc0, acc1, dinv_col,
                    b1[:128].reshape(1, 128), b1[128:].reshape(1, 128),
                    W23[:128], W23[128:])
    p0, p1 = _agg23(y23, row_d, col_d, ew_d)
    mu, logvar = _mmout_call(p0, p1, dinv_col,
                             b2.reshape(1, OUT), b3.reshape(1, OUT))

    adj = _adj_call(mu, mu)
    return (mu, logvar, mu, adj)


# speedup vs baseline: 1.5060x; 1.5060x over previous
import functools

import jax
import jax.numpy as jnp
from jax import lax
from jax.experimental import pallas as pl
from jax.experimental.pallas import tpu as pltpu
from jax.experimental.pallas import tpu_sc as plsc

N = 10000
E = 160000
FEAT = 256
HID = 256
OUT = 64

N_PAD = 10240
E_PAD = 163840
ROWS_PER_TILE = N // 16

_MESH = plsc.VectorSubcoreMesh(core_axis_name="c", subcore_axis_name="s")


def _deg_body(col_h, ew_h, out_h, idx_v, ew_v, zbuf, degacc):
    c = lax.axis_index("c")
    s = lax.axis_index("s")
    w = c * 16 + s
    for j in range(40):
        zbuf[pl.ds(16 * j, 16)] = jnp.zeros((16,), jnp.float32)
    pltpu.sync_copy(zbuf, degacc.at[pl.ds(640 * s, 640)])
    pltpu.sync_copy(col_h.at[w], idx_v)
    pltpu.sync_copy(ew_h.at[w], ew_v)
    plsc.subcore_barrier()

    def step(k, carry):
        pltpu.sync_copy(ew_v.at[k], degacc.at[idx_v.at[k]], add=True)
        return carry

    lax.fori_loop(0, 40, step, 0)
    plsc.subcore_barrier()
    pltpu.sync_copy(degacc.at[pl.ds(640 * s, 640)], out_h.at[c, pl.ds(640 * s, 640)])


_deg_kernel = pl.kernel(
    _deg_body,
    out_type=jax.ShapeDtypeStruct((2, N_PAD), jnp.float32),
    mesh=_MESH,
    scratch_types=[
        pltpu.VMEM((40, 128), jnp.int32),
        pltpu.VMEM((40, 128), jnp.float32),
        pltpu.VMEM((640,), jnp.float32),
        pltpu.VMEM_SHARED((N_PAD,), jnp.float32),
    ],
)


def _agg_body(D, ya, yb, row_h, col_h, ew_h, outa, outb,
              ridx, cidx, ewv, rows, rows2, acc, sem, sem2):
    c = lax.axis_index("c")
    s = lax.axis_index("s")
    base = 640 * s

    def _stripe_copy(src_at, dst_at):
        @pl.when(s < 15)
        def _():
            pltpu.sync_copy(src_at(base, 640), dst_at(base, 640))

        @pl.when(s == 15)
        def _():
            pltpu.sync_copy(src_at(9600, 400), dst_at(9600, 400))

    @pl.when(c == 0)
    def _():
        _stripe_copy(lambda b, n: ya.at[pl.ds(b, n)],
                     lambda b, n: acc.at[pl.ds(b, n)])

    @pl.when(c == 1)
    def _():
        _stripe_copy(lambda b, n: yb.at[pl.ds(b, n)],
                     lambda b, n: acc.at[pl.ds(b, n)])

    plsc.subcore_barrier()

    def _gather(k, rbuf, gsem):
        @pl.when(c == 0)
        def _():
            pltpu.async_copy(ya.at[ridx.at[k]], rbuf, gsem)

        @pl.when(c == 1)
        def _():
            pltpu.async_copy(yb.at[ridx.at[k]], rbuf, gsem)

    def _drain(rbuf, gsem):
        pltpu.make_async_copy(ya.at[pl.ds(0, 128)], rbuf, gsem).wait()

    def _scale(k, rbuf):
        def scale_group(g2, rc):
            for gg in range(2):
                g = 2 * g2 + gg
                ewvec = ewv[k, pl.ds(16 * g, 16)]
                for r in range(16):
                    wv = lax.gather(
                        ewvec, jnp.full((16, 1), r, jnp.int32),
                        lax.GatherDimensionNumbers(
                            offset_dims=(), collapsed_slice_dims=(0,),
                            start_index_map=(0,)),
                        (1,), mode=lax.GatherScatterMode.PROMISE_IN_BOUNDS)
                    row = 16 * g + r
                    for q in range(8):
                        rbuf[row, pl.ds(16 * q, 16)] = (
                            rbuf[row, pl.ds(16 * q, 16)] * wv)
            return rc

        lax.fori_loop(0, 4, scale_group, 0)

    for p in range(2):
        pltpu.sync_copy(row_h.at[s, pl.ds(40 * p, 40)], ridx)
        pltpu.sync_copy(col_h.at[s, pl.ds(40 * p, 40)], cidx)
        pltpu.sync_copy(ew_h.at[s, pl.ds(40 * p, 40)], ewv)
        _gather(0, rows, sem)
        _gather(1, rows2, sem2)

        def step(k, carry):
            for b, (rbuf, gsem) in enumerate(((rows, sem), (rows2, sem2))):
                j = 2 * k + b
                _drain(rbuf, gsem)
                _scale(j, rbuf)
                pltpu.sync_copy(rbuf, acc.at[cidx.at[j]], add=True)

                @pl.when(j + 2 < 40)
                def _():
                    _gather(j + 2, rbuf, gsem)
            return carry

        lax.fori_loop(0, 20, step, 0)
    plsc.subcore_barrier()

    @pl.when(c == 0)
    def _():
        _stripe_copy(lambda b, n: acc.at[pl.ds(b, n)],
                     lambda b, n: outa.at[pl.ds(b, n)])

    @pl.when(c == 1)
    def _():
        _stripe_copy(lambda b, n: acc.at[pl.ds(b, n)],
                     lambda b, n: outb.at[pl.ds(b, n)])


_agg128 = pl.kernel(
    functools.partial(_agg_body, 128),
    out_type=(jax.ShapeDtypeStruct((N, 128), jnp.float32),
              jax.ShapeDtypeStruct((N, 128), jnp.float32)),
    mesh=_MESH,
    scratch_types=[
        pltpu.VMEM((40, 128), jnp.int32),
        pltpu.VMEM((40, 128), jnp.int32),
        pltpu.VMEM((40, 128), jnp.float32),
        pltpu.VMEM((128, 128), jnp.float32),
        pltpu.VMEM((128, 128), jnp.float32),
        pltpu.VMEM_SHARED((N, 128), jnp.float32),
        pltpu.SemaphoreType.DMA,
        pltpu.SemaphoreType.DMA,
    ],
)


def _agg23_body(y23, row_h, col_h, ew_h, out0, out1,
                ridx, cidx, ewv, rows, rows2, acc, sem, sem2):
    c = lax.axis_index("c")
    s = lax.axis_index("s")
    w = c * 16 + s
    base = 640 * s

    def _stripe_copy(src_at, dst_at):
        @pl.when(s < 15)
        def _():
            pltpu.sync_copy(src_at(base, 640), dst_at(base, 640))

        @pl.when(s == 15)
        def _():
            pltpu.sync_copy(src_at(9600, 400), dst_at(9600, 400))

    @pl.when(c == 0)
    def _():
        _stripe_copy(lambda b, n: y23.at[pl.ds(b, n)],
                     lambda b, n: acc.at[pl.ds(b, n)])

    @pl.when(c == 1)
    def _():
        def zfill(j, carry):
            for q in range(8):
                rows[j, pl.ds(16 * q, 16)] = jnp.zeros((16,), jnp.float32)
            return carry

        lax.fori_loop(0, 128, zfill, 0)

        @pl.when(s < 15)
        def _():
            for t in range(5):
                pltpu.sync_copy(rows, acc.at[pl.ds(base + 128 * t, 128)])

        @pl.when(s == 15)
        def _():
            for t in range(3):
                pltpu.sync_copy(rows, acc.at[pl.ds(9600 + 128 * t, 128)])
            pltpu.sync_copy(rows.at[pl.ds(0, 16)], acc.at[pl.ds(9984, 16)])

    pltpu.sync_copy(row_h.at[w], ridx)
    pltpu.sync_copy(col_h.at[w], cidx)
    pltpu.sync_copy(ew_h.at[w], ewv)
    plsc.subcore_barrier()

    def _scale(k, rbuf):
        def scale_group(g2, rc):
            for gg in range(2):
                g = 2 * g2 + gg
                ewvec = ewv[k, pl.ds(16 * g, 16)]
                for r in range(16):
                    wv = lax.gather(
                        ewvec, jnp.full((16, 1), r, jnp.int32),
                        lax.GatherDimensionNumbers(
                            offset_dims=(), collapsed_slice_dims=(0,),
                            start_index_map=(0,)),
                        (1,), mode=lax.GatherScatterMode.PROMISE_IN_BOUNDS)
                    row = 16 * g + r
                    for q in range(8):
                        rbuf[row, pl.ds(16 * q, 16)] = (
                            rbuf[row, pl.ds(16 * q, 16)] * wv)
            return rc

        lax.fori_loop(0, 4, scale_group, 0)

    NB = 40
    pltpu.async_copy(y23.at[ridx.at[0]], rows, sem)
    pltpu.async_copy(y23.at[ridx.at[1]], rows2, sem2)

    def step(k, carry):
        for b, (rbuf, gsem) in enumerate(((rows, sem), (rows2, sem2))):
            j = 2 * k + b
            pltpu.make_async_copy(y23.at[pl.ds(0, 128)], rbuf, gsem).wait()
            _scale(j, rbuf)
            pltpu.sync_copy(rbuf, acc.at[cidx.at[j]], add=True)

            @pl.when(j + 2 < NB)
            def _():
                pltpu.async_copy(y23.at[ridx.at[j + 2]], rbuf, gsem)
        return carry

    lax.fori_loop(0, NB // 2, step, 0)
    plsc.subcore_barrier()

    @pl.when(c == 0)
    def _():
        _stripe_copy(lambda b, n: acc.at[pl.ds(b, n)],
                     lambda b, n: out0.at[pl.ds(b, n)])

    @pl.when(c == 1)
    def _():
        _stripe_copy(lambda b, n: acc.at[pl.ds(b, n)],
                     lambda b, n: out1.at[pl.ds(b, n)])


_agg23 = pl.kernel(
    _agg23_body,
    out_type=(jax.ShapeDtypeStruct((N, 128), jnp.float32),
              jax.ShapeDtypeStruct((N, 128), jnp.float32)),
    mesh=_MESH,
    scratch_types=[
        pltpu.VMEM((40, 128), jnp.int32),
        pltpu.VMEM((40, 128), jnp.int32),
        pltpu.VMEM((40, 128), jnp.float32),
        pltpu.VMEM((128, 128), jnp.float32),
        pltpu.VMEM((128, 128), jnp.float32),
        pltpu.VMEM_SHARED((N, 128), jnp.float32),
        pltpu.SemaphoreType.DMA,
        pltpu.SemaphoreType.DMA,
    ],
)


_PREC = lax.Precision.DEFAULT


def _dinv_body(deg_ref, out_ref):
    d = deg_ref[0] + deg_ref[1] + 1.0
    out_ref[...] = lax.rsqrt(d)


_dinv_call = pl.pallas_call(
    _dinv_body,
    out_shape=jax.ShapeDtypeStruct((N_PAD // 128, 128), jnp.float32),
)


def _mm1_body(x_ref, w_ref, dinv_ref, outa_ref, outb_ref):
    xw = lax.dot(x_ref[...], w_ref[...], precision=_PREC,
                 preferred_element_type=jnp.float32)
    y = xw * dinv_ref[...]
    outa_ref[...] = y[:, :128]
    outb_ref[...] = y[:, 128:]


_BM = 400

_mm1_call = pl.pallas_call(
    _mm1_body,
    grid=(N // _BM,),
    in_specs=[
        pl.BlockSpec((_BM, FEAT), lambda i: (i, 0)),
        pl.BlockSpec((FEAT, HID), lambda i: (0, 0)),
        pl.BlockSpec((_BM, 1), lambda i: (i, 0)),
    ],
    out_specs=[
        pl.BlockSpec((_BM, 128), lambda i: (i, 0)),
        pl.BlockSpec((_BM, 128), lambda i: (i, 0)),
    ],
    out_shape=[
        jax.ShapeDtypeStruct((N, 128), jnp.float32),
        jax.ShapeDtypeStruct((N, 128), jnp.float32),
    ],
)


def _mm2_body(acca_ref, accb_ref, dinv_ref, b1a_ref, b1b_ref,
              wa_ref, wb_ref, outa_ref):
    dinv = dinv_ref[...]
    x1a = jax.nn.sigmoid(acca_ref[...] * dinv + b1a_ref[...])
    x1b = jax.nn.sigmoid(accb_ref[...] * dinv + b1b_ref[...])
    y = lax.dot(x1a, wa_ref[...], precision=_PREC,
                preferred_element_type=jnp.float32)
    y = y + lax.dot(x1b, wb_ref[...], precision=_PREC,
                    preferred_element_type=jnp.float32)
    outa_ref[...] = y * dinv


_mm2_call = pl.pallas_call(
    _mm2_body,
    grid=(N // _BM,),
    in_specs=[
        pl.BlockSpec((_BM, 128), lambda i: (i, 0)),
        pl.BlockSpec((_BM, 128), lambda i: (i, 0)),
        pl.BlockSpec((_BM, 1), lambda i: (i, 0)),
        pl.BlockSpec((1, 128), lambda i: (0, 0)),
        pl.BlockSpec((1, 128), lambda i: (0, 0)),
        pl.BlockSpec((128, 2 * OUT), lambda i: (0, 0)),
        pl.BlockSpec((128, 2 * OUT), lambda i: (0, 0)),
    ],
    out_specs=pl.BlockSpec((_BM, 128), lambda i: (i, 0)),
    out_shape=jax.ShapeDtypeStruct((N, 128), jnp.float32),
)


def _mmout_body(p0_ref, p1_ref, dinv_ref, b2_ref, b3_ref,
                mu_ref, lv_ref):
    dinv = dinv_ref[...]
    su = p0_ref[...] + p1_ref[...]
    mu_ref[...] = jax.nn.sigmoid(su[:, :OUT] * dinv + b2_ref[...])
    lv_ref[...] = jax.nn.sigmoid(su[:, OUT:] * dinv + b3_ref[...])


_mmout_call = pl.pallas_call(
    _mmout_body,
    grid=(N // _BM,),
    in_specs=[
        pl.BlockSpec((_BM, 128), lambda i: (i, 0)),
        pl.BlockSpec((_BM, 128), lambda i: (i, 0)),
        pl.BlockSpec((_BM, 1), lambda i: (i, 0)),
        pl.BlockSpec((1, OUT), lambda i: (0, 0)),
        pl.BlockSpec((1, OUT), lambda i: (0, 0)),
    ],
    out_specs=[
        pl.BlockSpec((_BM, OUT), lambda i: (i, 0)),
        pl.BlockSpec((_BM, OUT), lambda i: (i, 0)),
    ],
    out_shape=[
        jax.ShapeDtypeStruct((N, OUT), jnp.float32),
        jax.ShapeDtypeStruct((N, OUT), jnp.float32),
    ],
)


def _adj_body(zi_ref, zj_ref, out_ref):
    p = lax.dot_general(zi_ref[...], zj_ref[...],
                        (((1,), (1,)), ((), ())),
                        precision=_PREC,
                        preferred_element_type=jnp.float32)
    out_ref[...] = jax.nn.sigmoid(p)


_BA = 512
_adj_call = pl.pallas_call(
    _adj_body,
    grid=(pl.cdiv(N, _BA), pl.cdiv(N, _BA)),
    in_specs=[
        pl.BlockSpec((_BA, OUT), lambda i, j: (i, 0)),
        pl.BlockSpec((_BA, OUT), lambda i, j: (j, 0)),
    ],
    out_specs=pl.BlockSpec((_BA, _BA), lambda i, j: (i, j)),
    out_shape=jax.ShapeDtypeStruct((N, N), jnp.float32),
)


def kernel(x, edge_index, edge_weight, W1, b1, W2, b2, W3, b3):
    row = edge_index[0].astype(jnp.int32)
    col = edge_index[1].astype(jnp.int32)
    ew = edge_weight.astype(jnp.float32)

    pad = E_PAD - E
    pad_idx = (jnp.arange(pad, dtype=jnp.int32) * 37) % N
    row_p = jnp.concatenate([row, pad_idx])
    col_p = jnp.concatenate([col, pad_idx])
    ew_p = jnp.concatenate([ew, jnp.zeros((pad,), jnp.float32)])

    row_d = row_p.reshape(32, 40, 128)
    col_d = col_p.reshape(32, 40, 128)
    ew_d = ew_p.reshape(32, 40, 128)
    row_a = row_p.reshape(16, 80, 128)
    col_a = col_p.reshape(16, 80, 128)
    ew_a = ew_p.reshape(16, 80, 128)

    deg_p = _deg_kernel(col_d, ew_d)
    dinv2d = _dinv_call(deg_p.reshape(2, N_PAD // 128, 128))
    dinv_col = dinv2d.reshape(N_PAD)[:N].reshape(N, 1)

    ya, yb = _mm1_call(x, W1, dinv_col)
    acc0, acc1 = _agg128(ya, yb, row_a, col_a, ew_a)

    W23 = jnp.concatenate([W2, W3], axis=1)
    y23 = _mm2_call(acc0, acc1, dinv_col,
                    b1[:128].reshape(1, 128), b1[128:].reshape(1, 128),
                    W23[:128], W23[128:])
    p0, p1 = _agg23(y23, row_d, col_d, ew_d)
    mu, logvar = _mmout_call(p0, p1, dinv_col,
                             b2.reshape(1, OUT), b3.reshape(1, OUT))

    adj = _adj_call(mu, mu)
    return (mu, logvar, mu, adj)


# speedup vs baseline: 1.5514x; 1.0301x over previous
import functools

import jax
import jax.numpy as jnp
from jax import lax
from jax.experimental import pallas as pl
from jax.experimental.pallas import tpu as pltpu
from jax.experimental.pallas import tpu_sc as plsc

N = 10000
E = 160000
FEAT = 256
HID = 256
OUT = 64

N_PAD = 10240
E_PAD = 163840
ROWS_PER_TILE = N // 16

_MESH = plsc.VectorSubcoreMesh(core_axis_name="c", subcore_axis_name="s")


def _deg_body(col_h, ew_h, out_h, idx_v, ew_v, zbuf, degacc):
    c = lax.axis_index("c")
    s = lax.axis_index("s")
    w = c * 16 + s
    for j in range(40):
        zbuf[pl.ds(16 * j, 16)] = jnp.zeros((16,), jnp.float32)
    pltpu.sync_copy(zbuf, degacc.at[pl.ds(640 * s, 640)])
    pltpu.sync_copy(col_h.at[w], idx_v)
    pltpu.sync_copy(ew_h.at[w], ew_v)
    plsc.subcore_barrier()

    def step(k, carry):
        pltpu.sync_copy(ew_v.at[k], degacc.at[idx_v.at[k]], add=True)
        return carry

    lax.fori_loop(0, 40, step, 0)
    plsc.subcore_barrier()
    pltpu.sync_copy(degacc.at[pl.ds(640 * s, 640)], out_h.at[c, pl.ds(640 * s, 640)])


_deg_kernel = pl.kernel(
    _deg_body,
    out_type=jax.ShapeDtypeStruct((2, N_PAD), jnp.float32),
    mesh=_MESH,
    scratch_types=[
        pltpu.VMEM((40, 128), jnp.int32),
        pltpu.VMEM((40, 128), jnp.float32),
        pltpu.VMEM((640,), jnp.float32),
        pltpu.VMEM_SHARED((N_PAD,), jnp.float32),
    ],
)


def _agg_body(D, ya, yb, row_h, col_h, ew_h, outa, outb,
              ridx, cidx, ewv, rows, rows2, acc, sem, sem2):
    c = lax.axis_index("c")
    s = lax.axis_index("s")
    base = 640 * s

    def _stripe_copy(src_at, dst_at):
        @pl.when(s < 15)
        def _():
            pltpu.sync_copy(src_at(base, 640), dst_at(base, 640))

        @pl.when(s == 15)
        def _():
            pltpu.sync_copy(src_at(9600, 400), dst_at(9600, 400))

    @pl.when(c == 0)
    def _():
        _stripe_copy(lambda b, n: ya.at[pl.ds(b, n)],
                     lambda b, n: acc.at[pl.ds(b, n)])

    @pl.when(c == 1)
    def _():
        _stripe_copy(lambda b, n: yb.at[pl.ds(b, n)],
                     lambda b, n: acc.at[pl.ds(b, n)])

    plsc.subcore_barrier()

    def _gather(k, rbuf, gsem):
        @pl.when(c == 0)
        def _():
            pltpu.async_copy(ya.at[ridx.at[k]], rbuf, gsem)

        @pl.when(c == 1)
        def _():
            pltpu.async_copy(yb.at[ridx.at[k]], rbuf, gsem)

    def _drain(rbuf, gsem):
        pltpu.make_async_copy(ya.at[pl.ds(0, 128)], rbuf, gsem).wait()

    def _scale(k, rbuf):
        def scale_group(g2, rc):
            for gg in range(2):
                g = 2 * g2 + gg
                ewvec = ewv[k, pl.ds(16 * g, 16)]
                for r in range(16):
                    wv = lax.gather(
                        ewvec, jnp.full((16, 1), r, jnp.int32),
                        lax.GatherDimensionNumbers(
                            offset_dims=(), collapsed_slice_dims=(0,),
                            start_index_map=(0,)),
                        (1,), mode=lax.GatherScatterMode.PROMISE_IN_BOUNDS)
                    row = 16 * g + r
                    for q in range(8):
                        rbuf[row, pl.ds(16 * q, 16)] = (
                            rbuf[row, pl.ds(16 * q, 16)] * wv)
            return rc

        lax.fori_loop(0, 4, scale_group, 0)

    for p in range(2):
        pltpu.sync_copy(row_h.at[s, pl.ds(40 * p, 40)], ridx)
        pltpu.sync_copy(col_h.at[s, pl.ds(40 * p, 40)], cidx)
        pltpu.sync_copy(ew_h.at[s, pl.ds(40 * p, 40)], ewv)
        _gather(0, rows, sem)
        _gather(1, rows2, sem2)

        def step(k, carry):
            for b, (rbuf, gsem) in enumerate(((rows, sem), (rows2, sem2))):
                j = 2 * k + b
                _drain(rbuf, gsem)
                _scale(j, rbuf)
                pltpu.sync_copy(rbuf, acc.at[cidx.at[j]], add=True)

                @pl.when(j + 2 < 40)
                def _():
                    _gather(j + 2, rbuf, gsem)
            return carry

        lax.fori_loop(0, 20, step, 0)
    plsc.subcore_barrier()

    @pl.when(c == 0)
    def _():
        _stripe_copy(lambda b, n: acc.at[pl.ds(b, n)],
                     lambda b, n: outa.at[pl.ds(b, n)])

    @pl.when(c == 1)
    def _():
        _stripe_copy(lambda b, n: acc.at[pl.ds(b, n)],
                     lambda b, n: outb.at[pl.ds(b, n)])


_agg128 = pl.kernel(
    functools.partial(_agg_body, 128),
    out_type=(jax.ShapeDtypeStruct((N, 128), jnp.float32),
              jax.ShapeDtypeStruct((N, 128), jnp.float32)),
    mesh=_MESH,
    scratch_types=[
        pltpu.VMEM((40, 128), jnp.int32),
        pltpu.VMEM((40, 128), jnp.int32),
        pltpu.VMEM((40, 128), jnp.float32),
        pltpu.VMEM((128, 128), jnp.float32),
        pltpu.VMEM((128, 128), jnp.float32),
        pltpu.VMEM_SHARED((N, 128), jnp.float32),
        pltpu.SemaphoreType.DMA,
        pltpu.SemaphoreType.DMA,
    ],
)


def _agg23_body(y23, row_h, col_h, ew_h, out0, out1,
                ridx, cidx, ewv, rows, rows2, acc, sem, sem2):
    c = lax.axis_index("c")
    s = lax.axis_index("s")
    w = c * 16 + s
    base = 640 * s

    def _stripe_copy(src_at, dst_at):
        @pl.when(s < 15)
        def _():
            pltpu.sync_copy(src_at(base, 640), dst_at(base, 640))

        @pl.when(s == 15)
        def _():
            pltpu.sync_copy(src_at(9600, 400), dst_at(9600, 400))

    @pl.when(c == 0)
    def _():
        _stripe_copy(lambda b, n: y23.at[pl.ds(b, n)],
                     lambda b, n: acc.at[pl.ds(b, n)])

    @pl.when(c == 1)
    def _():
        def zfill(j, carry):
            for q in range(8):
                rows[j, pl.ds(16 * q, 16)] = jnp.zeros((16,), jnp.float32)
            return carry

        lax.fori_loop(0, 128, zfill, 0)

        @pl.when(s < 15)
        def _():
            for t in range(5):
                pltpu.sync_copy(rows, acc.at[pl.ds(base + 128 * t, 128)])

        @pl.when(s == 15)
        def _():
            for t in range(3):
                pltpu.sync_copy(rows, acc.at[pl.ds(9600 + 128 * t, 128)])
            pltpu.sync_copy(rows.at[pl.ds(0, 16)], acc.at[pl.ds(9984, 16)])

    pltpu.sync_copy(row_h.at[w], ridx)
    pltpu.sync_copy(col_h.at[w], cidx)
    pltpu.sync_copy(ew_h.at[w], ewv)
    plsc.subcore_barrier()

    def _scale(k, rbuf):
        def scale_group(g2, rc):
            for gg in range(2):
                g = 2 * g2 + gg
                ewvec = ewv[k, pl.ds(16 * g, 16)]
                for r in range(16):
                    wv = lax.gather(
                        ewvec, jnp.full((16, 1), r, jnp.int32),
                        lax.GatherDimensionNumbers(
                            offset_dims=(), collapsed_slice_dims=(0,),
                            start_index_map=(0,)),
                        (1,), mode=lax.GatherScatterMode.PROMISE_IN_BOUNDS)
                    row = 16 * g + r
                    for q in range(8):
                        rbuf[row, pl.ds(16 * q, 16)] = (
                            rbuf[row, pl.ds(16 * q, 16)] * wv)
            return rc

        lax.fori_loop(0, 4, scale_group, 0)

    NB = 40
    pltpu.async_copy(y23.at[ridx.at[0]], rows, sem)
    pltpu.async_copy(y23.at[ridx.at[1]], rows2, sem2)

    def step(k, carry):
        for b, (rbuf, gsem) in enumerate(((rows, sem), (rows2, sem2))):
            j = 2 * k + b
            pltpu.make_async_copy(y23.at[pl.ds(0, 128)], rbuf, gsem).wait()
            _scale(j, rbuf)
            pltpu.sync_copy(rbuf, acc.at[cidx.at[j]], add=True)

            @pl.when(j + 2 < NB)
            def _():
                pltpu.async_copy(y23.at[ridx.at[j + 2]], rbuf, gsem)
        return carry

    lax.fori_loop(0, NB // 2, step, 0)
    plsc.subcore_barrier()

    @pl.when(c == 0)
    def _():
        _stripe_copy(lambda b, n: acc.at[pl.ds(b, n)],
                     lambda b, n: out0.at[pl.ds(b, n)])

    @pl.when(c == 1)
    def _():
        _stripe_copy(lambda b, n: acc.at[pl.ds(b, n)],
                     lambda b, n: out1.at[pl.ds(b, n)])


_agg23 = pl.kernel(
    _agg23_body,
    out_type=(jax.ShapeDtypeStruct((N, 128), jnp.float32),
              jax.ShapeDtypeStruct((N, 128), jnp.float32)),
    mesh=_MESH,
    scratch_types=[
        pltpu.VMEM((40, 128), jnp.int32),
        pltpu.VMEM((40, 128), jnp.int32),
        pltpu.VMEM((40, 128), jnp.float32),
        pltpu.VMEM((128, 128), jnp.float32),
        pltpu.VMEM((128, 128), jnp.float32),
        pltpu.VMEM_SHARED((N, 128), jnp.float32),
        pltpu.SemaphoreType.DMA,
        pltpu.SemaphoreType.DMA,
    ],
)


_PREC = lax.Precision.DEFAULT


def _dinv_body(deg_ref, out_ref):
    d = deg_ref[0] + deg_ref[1] + 1.0
    out_ref[...] = lax.rsqrt(d)


_dinv_call = pl.pallas_call(
    _dinv_body,
    out_shape=jax.ShapeDtypeStruct((N_PAD // 128, 128), jnp.float32),
)


def _mm1_body(x_ref, w_ref, dinv_ref, outa_ref, outb_ref):
    xw = lax.dot(x_ref[...], w_ref[...], precision=_PREC,
                 preferred_element_type=jnp.float32)
    y = xw * dinv_ref[...]
    outa_ref[...] = y[:, :128]
    outb_ref[...] = y[:, 128:]


_BM = 400

_mm1_call = pl.pallas_call(
    _mm1_body,
    grid=(N // _BM,),
    in_specs=[
        pl.BlockSpec((_BM, FEAT), lambda i: (i, 0)),
        pl.BlockSpec((FEAT, HID), lambda i: (0, 0)),
        pl.BlockSpec((_BM, 1), lambda i: (i, 0)),
    ],
    out_specs=[
        pl.BlockSpec((_BM, 128), lambda i: (i, 0)),
        pl.BlockSpec((_BM, 128), lambda i: (i, 0)),
    ],
    out_shape=[
        jax.ShapeDtypeStruct((N, 128), jnp.float32),
        jax.ShapeDtypeStruct((N, 128), jnp.float32),
    ],
)


def _mm2_body(acca_ref, accb_ref, dinv_ref, b1a_ref, b1b_ref,
              wa_ref, wb_ref, outa_ref):
    dinv = dinv_ref[...]
    x1a = jax.nn.sigmoid(acca_ref[...] * dinv + b1a_ref[...])
    x1b = jax.nn.sigmoid(accb_ref[...] * dinv + b1b_ref[...])
    y = lax.dot(x1a, wa_ref[...], precision=_PREC,
                preferred_element_type=jnp.float32)
    y = y + lax.dot(x1b, wb_ref[...], precision=_PREC,
                    preferred_element_type=jnp.float32)
    outa_ref[...] = y * dinv


_mm2_call = pl.pallas_call(
    _mm2_body,
    grid=(N // _BM,),
    in_specs=[
        pl.BlockSpec((_BM, 128), lambda i: (i, 0)),
        pl.BlockSpec((_BM, 128), lambda i: (i, 0)),
        pl.BlockSpec((_BM, 1), lambda i: (i, 0)),
        pl.BlockSpec((1, 128), lambda i: (0, 0)),
        pl.BlockSpec((1, 128), lambda i: (0, 0)),
        pl.BlockSpec((128, 2 * OUT), lambda i: (0, 0)),
        pl.BlockSpec((128, 2 * OUT), lambda i: (0, 0)),
    ],
    out_specs=pl.BlockSpec((_BM, 128), lambda i: (i, 0)),
    out_shape=jax.ShapeDtypeStruct((N, 128), jnp.float32),
)


def _mmout_body(p0_ref, p1_ref, dinv_ref, b2_ref, b3_ref,
                mu_ref, lv_ref):
    dinv = dinv_ref[...]
    su = p0_ref[...] + p1_ref[...]
    mu_ref[...] = jax.nn.sigmoid(su[:, :OUT] * dinv + b2_ref[...])
    lv_ref[...] = jax.nn.sigmoid(su[:, OUT:] * dinv + b3_ref[...])


_mmout_call = pl.pallas_call(
    _mmout_body,
    grid=(N // _BM,),
    in_specs=[
        pl.BlockSpec((_BM, 128), lambda i: (i, 0)),
        pl.BlockSpec((_BM, 128), lambda i: (i, 0)),
        pl.BlockSpec((_BM, 1), lambda i: (i, 0)),
        pl.BlockSpec((1, OUT), lambda i: (0, 0)),
        pl.BlockSpec((1, OUT), lambda i: (0, 0)),
    ],
    out_specs=[
        pl.BlockSpec((_BM, OUT), lambda i: (i, 0)),
        pl.BlockSpec((_BM, OUT), lambda i: (i, 0)),
    ],
    out_shape=[
        jax.ShapeDtypeStruct((N, OUT), jnp.float32),
        jax.ShapeDtypeStruct((N, OUT), jnp.float32),
    ],
)


def _adj_body(zi_ref, zj_ref, out_ref):
    p = lax.dot_general(zi_ref[...], zj_ref[...],
                        (((1,), (1,)), ((), ())),
                        precision=_PREC,
                        preferred_element_type=jnp.float32)
    out_ref[...] = 0.5 + 0.5 * jnp.tanh(0.5 * p)


_BA = 512
_adj_call = pl.pallas_call(
    _adj_body,
    grid=(pl.cdiv(N, _BA), pl.cdiv(N, _BA)),
    in_specs=[
        pl.BlockSpec((_BA, OUT), lambda i, j: (i, 0)),
        pl.BlockSpec((_BA, OUT), lambda i, j: (j, 0)),
    ],
    out_specs=pl.BlockSpec((_BA, _BA), lambda i, j: (i, j)),
    out_shape=jax.ShapeDtypeStruct((N, N), jnp.float32),
)


def kernel(x, edge_index, edge_weight, W1, b1, W2, b2, W3, b3):
    row = edge_index[0].astype(jnp.int32)
    col = edge_index[1].astype(jnp.int32)
    ew = edge_weight.astype(jnp.float32)

    pad = E_PAD - E
    pad_idx = (jnp.arange(pad, dtype=jnp.int32) * 37) % N
    row_p = jnp.concatenate([row, pad_idx])
    col_p = jnp.concatenate([col, pad_idx])
    ew_p = jnp.concatenate([ew, jnp.zeros((pad,), jnp.float32)])

    row_d = row_p.reshape(32, 40, 128)
    col_d = col_p.reshape(32, 40, 128)
    ew_d = ew_p.reshape(32, 40, 128)
    row_a = row_p.reshape(16, 80, 128)
    col_a = col_p.reshape(16, 80, 128)
    ew_a = ew_p.reshape(16, 80, 128)

    deg_p = _deg_kernel(col_d, ew_d)
    dinv2d = _dinv_call(deg_p.reshape(2, N_PAD // 128, 128))
    dinv_col = dinv2d.reshape(N_PAD)[:N].reshape(N, 1)

    ya, yb = _mm1_call(x, W1, dinv_col)
    acc0, acc1 = _agg128(ya, yb, row_a, col_a, ew_a)

    W23 = jnp.concatenate([W2, W3], axis=1)
    y23 = _mm2_call(acc0, acc1, dinv_col,
                    b1[:128].reshape(1, 128), b1[128:].reshape(1, 128),
                    W23[:128], W23[128:])
    p0, p1 = _agg23(y23, row_d, col_d, ew_d)
    mu, logvar = _mmout_call(p0, p1, dinv_col,
                             b2.reshape(1, OUT), b3.reshape(1, OUT))

    adj = _adj_call(mu, mu)
    return (mu, logvar, mu, adj)
